# baseline (speedup 1.0000x reference)
import functools

import jax
import jax.numpy as jnp
from jax import lax
from jax.experimental import pallas as pl
from jax.experimental.pallas import tpu as pltpu
from jax.experimental.pallas import tpu_sc as plsc

_B = 16384
_D = 64

_info = plsc.get_sparse_core_info()
_NC = _info.num_cores
_NS = _info.num_subcores
_L = _info.num_lanes
_NW = _NC * _NS
_BPW = _B // _NW
_CHUNK = 128
_NCHUNK = _BPW // _CHUNK

_mesh = plsc.VectorSubcoreMesh(core_axis_name="c", subcore_axis_name="s")


@functools.partial(
    pl.kernel,
    mesh=_mesh,
    compiler_params=pltpu.CompilerParams(needs_layout_passes=False,
                                         use_tc_tiling_on_sc=False),
    out_type=jax.ShapeDtypeStruct((_NW, _L), jnp.float32),
    scratch_types=[
        pltpu.VMEM((_NCHUNK, _CHUNK), jnp.int32),
        pltpu.VMEM((_BPW, _D), jnp.float32),
        pltpu.VMEM((_BPW, _D), jnp.float32),
        pltpu.VMEM((_L,), jnp.float32),
        pltpu.SemaphoreType.DMA,
        pltpu.SemaphoreType.DMA,
    ],
)
def _center_loss_partials(x_hbm, labels_hbm, centers_hbm, out_hbm,
                          idx_v, x_v, c_v, tot_v, gsem, xsem):
    wid = lax.axis_index("s") * _NC + lax.axis_index("c")
    base = wid * _BPW

    xcopy = pltpu.async_copy(x_hbm.at[pl.ds(base, _BPW)], x_v, xsem)
    for j in range(_NCHUNK):
        pltpu.sync_copy(labels_hbm.at[pl.ds(base + j * _CHUNK, _CHUNK)],
                        idx_v.at[j])
    gathers = [
        pltpu.async_copy(centers_hbm.at[idx_v.at[j]],
                         c_v.at[pl.ds(j * _CHUNK, _CHUNK)], gsem)
        for j in range(_NCHUNK)
    ]
    for g in gathers:
        g.wait()
    xcopy.wait()

    def row_body(r, tot):
        acc = jnp.zeros((_L,), jnp.float32)
        for k in range(_D // _L):
            xa = x_v[r, pl.ds(k * _L, _L)]
            ca = c_v[r, pl.ds(k * _L, _L)]
            dd = xa - ca
            acc = acc + dd * dd
        dist = jnp.sum(acc)
        dist = jnp.minimum(jnp.maximum(dist, 1e-12), 1e12)
        return tot + dist

    tot = lax.fori_loop(0, _BPW, row_body, jnp.float32(0.0))
    iota = lax.iota(jnp.int32, _L)
    tot_v[...] = jnp.where(iota < 1, tot, jnp.float32(0.0))
    pltpu.sync_copy(tot_v, out_hbm.at[wid])


def kernel(x, labels, centers):
    partials = _center_loss_partials(x, labels.astype(jnp.int32), centers)
    return jnp.sum(partials) * (1.0 / _B)


# speedup vs baseline: 1.1500x; 1.1500x over previous
import functools

import jax
import jax.numpy as jnp
from jax import lax
from jax.experimental import pallas as pl
from jax.experimental.pallas import tpu as pltpu
from jax.experimental.pallas import tpu_sc as plsc

_B = 16384
_D = 64
_C = 100000

_info = plsc.get_sparse_core_info()
_NC = _info.num_cores
_NS = _info.num_subcores
_L = _info.num_lanes
_NW = _NC * _NS
_BPW = _B // _NW
_CH = 32
_NPAIR = _BPW // (2 * _CH)

_mesh = plsc.VectorSubcoreMesh(core_axis_name="c", subcore_axis_name="s")


@functools.partial(
    pl.kernel,
    mesh=_mesh,
    compiler_params=pltpu.CompilerParams(needs_layout_passes=False),
    out_type=jax.ShapeDtypeStruct((_NW * _L,), jnp.float32),
    scratch_types=[
        pltpu.VMEM((_BPW,), jnp.int32),
        pltpu.VMEM((_D, _BPW), jnp.float32),
        pltpu.VMEM((2, _CH, 8, _D), jnp.float32),
        pltpu.VMEM((_L,), jnp.float32),
        pltpu.SemaphoreType.DMA,
        pltpu.SemaphoreType.DMA,
        pltpu.SemaphoreType.DMA,
    ],
)
def _center_loss_partials(xt_hbm, labels_hbm, centers_hbm, out_hbm,
                          lbl_v, x_v, c_v, tot_v, semA, semB, xsem):
    wid = lax.axis_index("s") * _NC + lax.axis_index("c")
    base = wid * _BPW

    pltpu.sync_copy(labels_hbm.at[pl.ds(base, _BPW)], lbl_v)
    xcopy = pltpu.async_copy(xt_hbm.at[:, pl.ds(base, _BPW)], x_v, xsem)

    def fire(k, buf, sem):
        for g in range(_CH // _L):
            lblv = lbl_v[pl.ds(k * _CH + g * _L, _L)]
            tv = lblv >> 3
            for i in range(_L):
                pltpu.async_copy(centers_hbm.at[tv[i]],
                                 buf.at[g * _L + i], sem)

    def drain(buf, sem):
        pltpu.make_async_copy(centers_hbm.at[pl.ds(0, _CH)], buf, sem).wait()

    iotas = [kk * _L + lax.iota(jnp.int32, _L) for kk in range(_D // _L)]

    def comp(k, buf, tot):
        for g in range(_CH // _L):
            r0 = k * _CH + g * _L
            lblv = lbl_v[pl.ds(r0, _L)]
            sv = lblv & 7
            for i in range(_L):
                col = jnp.broadcast_to(r0 + i, (_L,)).astype(jnp.int32)
                acc = jnp.zeros((_L,), jnp.float32)
                for kk in range(_D // _L):
                    xa = plsc.load_gather(x_v, [iotas[kk], col])
                    ca = buf[g * _L + i, sv[i], pl.ds(kk * _L, _L)]
                    dd = xa - ca
                    acc = acc + dd * dd
                dist = jnp.sum(acc)
                dist = jnp.minimum(jnp.maximum(dist, 1e-12), 1e12)
                tot = tot + dist
        return tot

    fire(0, c_v.at[0], semA)
    xcopy.wait()

    def pair_body(m, tot):
        fire(2 * m + 1, c_v.at[1], semB)
        drain(c_v.at[0], semA)
        tot = comp(2 * m, c_v.at[0], tot)

        @pl.when(m < _NPAIR - 1)
        def _():
            fire(2 * m + 2, c_v.at[0], semA)

        drain(c_v.at[1], semB)
        return comp(2 * m + 1, c_v.at[1], tot)

    tot = lax.fori_loop(0, _NPAIR, pair_body, jnp.float32(0.0))
    iota = lax.iota(jnp.int32, _L)
    tot_v[...] = jnp.where(iota < 1, tot, jnp.float32(0.0))
    pltpu.sync_copy(tot_v, out_hbm.at[pl.ds(wid * _L, _L)])


def kernel(x, labels, centers):
    centers3 = centers.reshape(_C // 8, 8, _D)
    partials = _center_loss_partials(x.T, labels.astype(jnp.int32), centers3)
    return jnp.sum(partials) * (1.0 / _B)
